# baseline (speedup 1.0000x reference)
import functools

import jax
import jax.numpy as jnp
from jax import lax
from jax.experimental import pallas as pl
from jax.experimental.pallas import tpu as pltpu
from jax.experimental.pallas import tpu_sc as plsc

N = 10000
E = 320000
D = 128
OUT = 16
G = 64
EPS = 1e-5

NC, NS = 2, 16
NW = NC * NS
EW_PER = E // NW
CHUNK = 80
NCHUNK = EW_PER // CHUNK
WB_TILES = 10
WB_ROWS = N // WB_TILES
LANES = 16

_mesh = plsc.VectorSubcoreMesh(core_axis_name="c", subcore_axis_name="s",
                               num_cores=NC, num_subcores=NS)


_sc_params = pltpu.CompilerParams(use_tc_tiling_on_sc=False)


@functools.partial(
    pl.kernel,
    out_type=jax.ShapeDtypeStruct((NC * N,), jnp.float32),
    mesh=_mesh,
    scratch_types=[
        pltpu.VMEM((NCHUNK, CHUNK), jnp.int32),
        pltpu.VMEM((NCHUNK, CHUNK), jnp.float32),
        pltpu.VMEM((2000,), jnp.float32),
        pltpu.VMEM((NCHUNK + 3,), jnp.int32),
        pltpu.VMEM_SHARED((N,), jnp.float32),
        pltpu.SemaphoreType.DMA,
    ],
    compiler_params=_sc_params,
)
def _deg_kernel(col_hbm, ew_hbm, out_hbm, col_v, ew_v, zbuf, idx_v, acc,
                dsem):
    c = lax.axis_index("c")
    s = lax.axis_index("s")
    wid = s * NC + c

    def ifill(i, carry):
        iv = (lax.broadcasted_iota(jnp.int32, (LANES,), 0)
              + (wid * NCHUNK + i * LANES))
        idx_v[pl.ds(i * LANES, LANES)] = jnp.minimum(iv, NW * NCHUNK - 1)
        return carry

    lax.fori_loop(0, (NCHUNK + 3) // LANES, ifill, 0)
    idx125 = idx_v.at[pl.ds(0, NCHUNK)]
    pltpu.async_copy(col_hbm.at[idx125], col_v, dsem).wait()
    pltpu.async_copy(ew_hbm.at[idx125], ew_v, dsem).wait()

    def zf(i, carry):
        zbuf[pl.ds(i * LANES, LANES)] = jnp.zeros((LANES,), jnp.float32)
        return carry

    lax.fori_loop(0, 2000 // LANES, zf, 0)

    @pl.when(s < 5)
    def _():
        pltpu.sync_copy(zbuf, acc.at[pl.ds(s * 2000, 2000)])

    plsc.subcore_barrier()

    def burst(b, carry):
        def fire(k, carry2):
            pltpu.async_copy(ew_v.at[b * 25 + k], acc.at[col_v.at[b * 25 + k]],
                             dsem, add=True)
            return carry2

        lax.fori_loop(0, 25, fire, 0)

        def drain(k, carry2):
            pltpu.make_async_copy(ew_v.at[0], acc.at[col_v.at[0]],
                                  dsem).wait()
            return carry2

        lax.fori_loop(0, 25, drain, 0)
        return carry

    lax.fori_loop(0, NCHUNK // 25, burst, 0)
    plsc.subcore_barrier()

    @pl.when(s < 5)
    def _():
        pltpu.sync_copy(acc.at[pl.ds(s * 2000, 2000)], zbuf)
        pltpu.sync_copy(zbuf, out_hbm.at[pl.ds(c * N + s * 2000, 2000)])


@functools.partial(
    pl.kernel,
    out_type=jax.ShapeDtypeStruct((NC * N, D), jnp.float32),
    mesh=_mesh,
    scratch_types=[
        pltpu.VMEM((NCHUNK, CHUNK), jnp.int32),
        pltpu.VMEM((2 * NCHUNK, CHUNK // 2), jnp.int32),
        pltpu.VMEM((NCHUNK, CHUNK), jnp.float32),
        pltpu.VMEM((CHUNK, D), jnp.float32),
        pltpu.VMEM((CHUNK, D), jnp.float32),
        pltpu.VMEM((NCHUNK + 3,), jnp.int32),
        pltpu.VMEM((2 * NCHUNK + 6,), jnp.int32),
        pltpu.VMEM_SHARED((N, D), jnp.float32),
        pltpu.SemaphoreType.DMA,
        pltpu.SemaphoreType.DMA,
        pltpu.SemaphoreType.DMA,
        pltpu.SemaphoreType.DMA,
    ],
    compiler_params=_sc_params,
)
def _scat_kernel(hs_hbm, row_hbm, col_hbm, ew_hbm, out_hbm,
                 row_v, col_v, ew_v, gbufA, gbufB, idx_v, idx2_v, acc,
                 gsemA, gsemB, ssemA, ssemB):
    c = lax.axis_index("c")
    s = lax.axis_index("s")
    wid = s * NC + c

    def ifill(i, carry):
        iv = (lax.broadcasted_iota(jnp.int32, (LANES,), 0)
              + (wid * NCHUNK + i * LANES))
        idx_v[pl.ds(i * LANES, LANES)] = jnp.minimum(iv, NW * NCHUNK - 1)
        return carry

    lax.fori_loop(0, (NCHUNK + 3) // LANES, ifill, 0)

    def ifill2(i, carry):
        iv = (lax.broadcasted_iota(jnp.int32, (LANES,), 0)
              + (wid * 2 * NCHUNK + i * LANES))
        idx2_v[pl.ds(i * LANES, LANES)] = jnp.minimum(iv, NW * 2 * NCHUNK - 1)
        return carry

    lax.fori_loop(0, (2 * NCHUNK + 6) // LANES, ifill2, 0)
    idx125 = idx_v.at[pl.ds(0, NCHUNK)]
    idx250 = idx2_v.at[pl.ds(0, 2 * NCHUNK)]
    pltpu.async_copy(row_hbm.at[idx125], row_v, gsemA).wait()
    pltpu.async_copy(hs_hbm.at[row_v.at[0]], gbufA, gsemA)
    pltpu.async_copy(col_hbm.at[idx250], col_v, gsemB)
    pltpu.async_copy(ew_hbm.at[idx125], ew_v, ssemA)

    def zf(r, carry):
        for cc in range(D // LANES):
            gbufB[r, pl.ds(cc * LANES, LANES)] = jnp.zeros((LANES,),
                                                           jnp.float32)
        return carry

    lax.fori_loop(0, CHUNK, zf, 0)

    @pl.when(s < WB_TILES)
    def _():
        def zc(m, carry):
            pltpu.async_copy(gbufB, acc.at[pl.ds(s * WB_ROWS + m * CHUNK,
                                                 CHUNK)], ssemB)
            return carry

        lax.fori_loop(0, WB_ROWS // CHUNK, zc, 0)
        pltpu.async_copy(gbufB.at[pl.ds(0, 40)],
                         acc.at[pl.ds(s * WB_ROWS + 960, 40)], ssemB)

        def zd(m, carry):
            pltpu.make_async_copy(
                gbufB, acc.at[pl.ds(s * WB_ROWS, CHUNK)], ssemB).wait()
            return carry

        lax.fori_loop(0, WB_ROWS // CHUNK, zd, 0)
        pltpu.make_async_copy(gbufB.at[pl.ds(0, 40)],
                              acc.at[pl.ds(s * WB_ROWS, 40)], ssemB).wait()

    pltpu.make_async_copy(col_hbm.at[idx125], col_v, gsemB).wait()
    pltpu.make_async_copy(ew_hbm.at[idx125], ew_v, ssemA).wait()
    plsc.subcore_barrier()

    dn = lax.GatherDimensionNumbers(
        offset_dims=(), collapsed_slice_dims=(0,), start_index_map=(0,))

    H = CHUNK // 2

    def scale_block(buf, wch, r0):
        for l in range(LANES):
            wv = lax.gather(wch, jnp.full((LANES, 1), l, jnp.int32), dn,
                            slice_sizes=(1,),
                            mode=lax.GatherScatterMode.PROMISE_IN_BOUNDS)
            r = r0 + l
            for cc in range(D // LANES):
                sl = pl.ds(cc * LANES, LANES)
                buf[r, sl] = buf[r, sl] * wv

    def scale_tail(buf, wch, r0):
        for l in range(8):
            wv = lax.gather(wch, jnp.full((LANES, 1), l + 8, jnp.int32), dn,
                            slice_sizes=(1,),
                            mode=lax.GatherScatterMode.PROMISE_IN_BOUNDS)
            r = r0 + l
            for cc in range(D // LANES):
                sl = pl.ds(cc * LANES, LANES)
                buf[r, sl] = buf[r, sl] * wv

    def process(buf, j, ssem):
        for h in range(2):
            b = h * H
            scale_block(buf, ew_v[j, pl.ds(b, LANES)], b)
            scale_block(buf, ew_v[j, pl.ds(b + LANES, LANES)], b + LANES)
            scale_tail(buf, ew_v[j, pl.ds(b + 2 * LANES - 8, LANES)],
                       b + 2 * LANES)
            pltpu.async_copy(buf.at[pl.ds(b, H)], acc.at[col_v.at[2 * j + h]],
                             ssem, add=True)

    def gather_start(j, buf, sem):
        return pltpu.async_copy(hs_hbm.at[row_v.at[j]], buf, sem)

    def gather_wait(buf, sem):
        pltpu.make_async_copy(hs_hbm.at[row_v.at[0]], buf, sem).wait()

    def scat_wait(buf, sem):
        pltpu.make_async_copy(buf, acc.at[col_v.at[0]], sem).wait()


    def body(m, carry):
        jA = 2 * m
        jB = 2 * m + 1

        @pl.when(m > 0)
        def _():
            scat_wait(gbufB, ssemB)

        gather_start(jB, gbufB, gsemB)
        gather_wait(gbufA, gsemA)
        process(gbufA, jA, ssemA)
        gather_wait(gbufB, gsemB)
        process(gbufB, jB, ssemB)
        scat_wait(gbufA, ssemA)
        gather_start(jA + 2, gbufA, gsemA)
        return carry

    lax.fori_loop(0, (NCHUNK - 1) // 2, body, 0)

    gather_wait(gbufA, gsemA)
    process(gbufA, NCHUNK - 1, ssemA)
    scat_wait(gbufB, ssemB)
    scat_wait(gbufA, ssemA)

    plsc.subcore_barrier()

    @pl.when(s < WB_TILES)
    def _():
        pltpu.sync_copy(acc.at[pl.ds(s * WB_ROWS, WB_ROWS)],
                        out_hbm.at[pl.ds(c * N + s * WB_ROWS, WB_ROWS)])


BN = 2000
NB = N // BN
_seq = pltpu.CompilerParams(dimension_semantics=("arbitrary",))


def _tcA(deg_a, deg_b, x, W):
    def body(da_ref, db_ref, x_ref, w_ref, dis_ref, hs_ref):
        deg = da_ref[0, 0, :] + db_ref[0, 0, :] + 1.0
        dis = jnp.where(deg > 0, lax.rsqrt(jnp.maximum(deg, 1e-12)), 0.0)
        dis_ref[0, 0, :] = dis
        h = jnp.dot(x_ref[...], w_ref[...], preferred_element_type=jnp.float32)
        hs_ref[...] = h * dis[:, None]

    return pl.pallas_call(
        body,
        grid=(NB,),
        in_specs=[
            pl.BlockSpec((1, 1, BN), lambda i: (i, 0, 0)),
            pl.BlockSpec((1, 1, BN), lambda i: (i, 0, 0)),
            pl.BlockSpec((BN, D), lambda i: (i, 0)),
            pl.BlockSpec((D, D), lambda i: (0, 0)),
        ],
        out_specs=[
            pl.BlockSpec((1, 1, BN), lambda i: (i, 0, 0)),
            pl.BlockSpec((BN, D), lambda i: (i, 0)),
        ],
        out_shape=[
            jax.ShapeDtypeStruct((NB, 1, BN), jnp.float32),
            jax.ShapeDtypeStruct((N, D), jnp.float32),
        ],
        compiler_params=_seq,
    )(deg_a, deg_b, x, W)


def _tcB(sa, sb, hs0, dis, b, gamma, beta, W):
    def body(sa_ref, sb_ref, hs_ref, dis_ref, b_ref, g_ref, be_ref, w_ref,
             out_ref):
        dis_v = dis_ref[0, 0, :]
        t = (sa_ref[...] + sb_ref[...] + hs_ref[...]) * dis_v[:, None]
        t = t + b_ref[0, :][None, :]
        t = jnp.maximum(t, 0.0)
        t = t * (g_ref[0, :] / jnp.sqrt(1.0 + EPS))[None, :] + be_ref[0, :][None, :]
        h1 = jnp.dot(t, w_ref[...], preferred_element_type=jnp.float32)
        out_ref[...] = h1 * dis_v[:, None]

    return pl.pallas_call(
        body,
        grid=(NB,),
        in_specs=[
            pl.BlockSpec((BN, D), lambda i: (i, 0)),
            pl.BlockSpec((BN, D), lambda i: (i, 0)),
            pl.BlockSpec((BN, D), lambda i: (i, 0)),
            pl.BlockSpec((1, 1, BN), lambda i: (i, 0, 0)),
            pl.BlockSpec((1, D), lambda i: (0, 0)),
            pl.BlockSpec((1, D), lambda i: (0, 0)),
            pl.BlockSpec((1, D), lambda i: (0, 0)),
            pl.BlockSpec((D, D), lambda i: (0, 0)),
        ],
        out_specs=pl.BlockSpec((BN, D), lambda i: (i, 0)),
        out_shape=jax.ShapeDtypeStruct((N, D), jnp.float32),
        compiler_params=_seq,
    )(sa, sb, hs0, dis, b, gamma, beta, W)


def _tcC(sa, sb, hs1, dis, b, gamma, beta, w1, b1, w2, b2, batch,
         cw1, cb1, cw2, cb2):
    def body(sa_ref, sb_ref, hs_ref, dis_ref, b_ref, g_ref, be_ref,
             w1_ref, b1_ref, w2_ref, b2_ref, batch_ref,
             cw1_ref, cb1_ref, cw2_ref, cb2_ref,
             h_ref, pool_ref, reps_ref, log_ref):
        i = pl.program_id(0)
        dis_v = dis_ref[0, 0, :]
        t = (sa_ref[...] + sb_ref[...] + hs_ref[...]) * dis_v[:, None]
        t = t + b_ref[0, :][None, :]
        t = jnp.maximum(t, 0.0)
        t = t * (g_ref[0, :] / jnp.sqrt(1.0 + EPS))[None, :] + be_ref[0, :][None, :]
        t1 = jnp.dot(t, w1_ref[...], preferred_element_type=jnp.float32)
        t1 = jnp.maximum(t1 + b1_ref[0, :][None, :], 0.0)
        ho = jnp.dot(t1, w2_ref[...], preferred_element_type=jnp.float32)
        ho = ho + b2_ref[0, :][None, :]
        h_ref[...] = ho
        seg_ids = lax.broadcasted_iota(jnp.int32, (G, BN), 0)
        M = (batch_ref[0, 0, :][None, :] == seg_ids).astype(jnp.float32)
        num = jnp.dot(M, ho, preferred_element_type=jnp.float32)
        cnt = jnp.sum(M, axis=1)
        blk = jnp.concatenate(
            [num, jnp.broadcast_to(cnt[:, None], (G, D))], axis=0)

        @pl.when(i == 0)
        def _():
            pool_ref[...] = jnp.zeros((2 * G, D), jnp.float32)

        pool_ref[...] += blk

        @pl.when(i == NB - 1)
        def _():
            reps = pool_ref[0:G, :] / jnp.maximum(pool_ref[G:2 * G, :], 1.0)
            reps_ref[...] = reps
            z = jnp.dot(reps, cw1_ref[...], preferred_element_type=jnp.float32)
            z = jnp.maximum(z + cb1_ref[0, :][None, :], 0.0)
            lg = jnp.dot(z, cw2_ref[...], preferred_element_type=jnp.float32)
            log_ref[...] = lg + cb2_ref[0, :][None, :]

    return pl.pallas_call(
        body,
        grid=(NB,),
        in_specs=[
            pl.BlockSpec((BN, D), lambda i: (i, 0)),
            pl.BlockSpec((BN, D), lambda i: (i, 0)),
            pl.BlockSpec((BN, D), lambda i: (i, 0)),
            pl.BlockSpec((1, 1, BN), lambda i: (i, 0, 0)),
            pl.BlockSpec((1, D), lambda i: (0, 0)),
            pl.BlockSpec((1, D), lambda i: (0, 0)),
            pl.BlockSpec((1, D), lambda i: (0, 0)),
            pl.BlockSpec((D, D), lambda i: (0, 0)),
            pl.BlockSpec((1, D), lambda i: (0, 0)),
            pl.BlockSpec((D, D), lambda i: (0, 0)),
            pl.BlockSpec((1, D), lambda i: (0, 0)),
            pl.BlockSpec((1, 1, BN), lambda i: (i, 0, 0)),
            pl.BlockSpec((D, D), lambda i: (0, 0)),
            pl.BlockSpec((1, D), lambda i: (0, 0)),
            pl.BlockSpec((D, OUT), lambda i: (0, 0)),
            pl.BlockSpec((1, OUT), lambda i: (0, 0)),
        ],
        out_specs=[
            pl.BlockSpec((BN, D), lambda i: (i, 0)),
            pl.BlockSpec((2 * G, D), lambda i: (0, 0)),
            pl.BlockSpec((G, D), lambda i: (0, 0)),
            pl.BlockSpec((G, OUT), lambda i: (0, 0)),
        ],
        out_shape=[
            jax.ShapeDtypeStruct((N, D), jnp.float32),
            jax.ShapeDtypeStruct((2 * G, D), jnp.float32),
            jax.ShapeDtypeStruct((G, D), jnp.float32),
            jax.ShapeDtypeStruct((G, OUT), jnp.float32),
        ],
        compiler_params=_seq,
    )(sa, sb, hs1, dis, b, gamma, beta, w1, b1, w2, b2, batch,
      cw1, cb1, cw2, cb2)


def kernel(x, edge_index, edge_attr, batch, W_gc0, b_gc0, gamma0, beta0,
           W_gc1, b_gc1, gamma1, beta1, lin1_W, lin1_b, lin2_W, lin2_b,
           cls1_W, cls1_b, cls2_W, cls2_b):
    ew = edge_attr[:, 0]
    row2 = edge_index[0].reshape(E // CHUNK, CHUNK)
    col2 = edge_index[1].reshape(E // CHUNK, CHUNK)
    colh = edge_index[1].reshape(E // (CHUNK // 2), CHUNK // 2)
    ew2 = ew.reshape(E // CHUNK, CHUNK)

    deg2 = _deg_kernel(col2, ew2).reshape(NC, N)
    dis3, hs0 = _tcA(deg2[0].reshape(NB, 1, BN), deg2[1].reshape(NB, 1, BN),
                     x, W_gc0)
    S0 = _scat_kernel(hs0, row2, colh, ew2)
    hs1 = _tcB(S0[:N], S0[N:], hs0, dis3, b_gc0.reshape(1, D),
               gamma0.reshape(1, D), beta0.reshape(1, D), W_gc1)
    S1 = _scat_kernel(hs1, row2, colh, ew2)
    hout, _pool, reps, logits = _tcC(
        S1[:N], S1[N:], hs1, dis3, b_gc1.reshape(1, D),
        gamma1.reshape(1, D), beta1.reshape(1, D),
        lin1_W, lin1_b.reshape(1, D), lin2_W,
        lin2_b.reshape(1, D), batch.reshape(NB, 1, BN),
        cls1_W, cls1_b.reshape(1, D), cls2_W, cls2_b.reshape(1, OUT))
    return (hout, reps, logits)


# speedup vs baseline: 1.3902x; 1.3902x over previous
import functools

import jax
import jax.numpy as jnp
from jax import lax
from jax.experimental import pallas as pl
from jax.experimental.pallas import tpu as pltpu
from jax.experimental.pallas import tpu_sc as plsc

N = 10000
E = 320000
D = 128
OUT = 16
G = 64
EPS = 1e-5

NC, NS = 2, 16
NW = NC * NS
EW_PER = E // NW
CHUNK = 80
NCHUNK = EW_PER // CHUNK
WB_TILES = 10
WB_ROWS = N // WB_TILES
LANES = 16

_mesh = plsc.VectorSubcoreMesh(core_axis_name="c", subcore_axis_name="s",
                               num_cores=NC, num_subcores=NS)


_sc_params = pltpu.CompilerParams(use_tc_tiling_on_sc=False)


@functools.partial(
    pl.kernel,
    out_type=jax.ShapeDtypeStruct((NC * N,), jnp.float32),
    mesh=_mesh,
    scratch_types=[
        pltpu.VMEM((NCHUNK, CHUNK), jnp.int32),
        pltpu.VMEM((NCHUNK, CHUNK), jnp.float32),
        pltpu.VMEM((2000,), jnp.float32),
        pltpu.VMEM((NCHUNK + 3,), jnp.int32),
        pltpu.VMEM_SHARED((N,), jnp.float32),
        pltpu.SemaphoreType.DMA,
    ],
    compiler_params=_sc_params,
)
def _deg_kernel(col_hbm, ew_hbm, out_hbm, col_v, ew_v, zbuf, idx_v, acc,
                dsem):
    c = lax.axis_index("c")
    s = lax.axis_index("s")
    wid = s * NC + c

    def ifill(i, carry):
        iv = (lax.broadcasted_iota(jnp.int32, (LANES,), 0)
              + (wid * NCHUNK + i * LANES))
        idx_v[pl.ds(i * LANES, LANES)] = jnp.minimum(iv, NW * NCHUNK - 1)
        return carry

    lax.fori_loop(0, (NCHUNK + 3) // LANES, ifill, 0)
    idx125 = idx_v.at[pl.ds(0, NCHUNK)]
    pltpu.async_copy(col_hbm.at[idx125], col_v, dsem).wait()
    pltpu.async_copy(ew_hbm.at[idx125], ew_v, dsem).wait()

    def zf(i, carry):
        zbuf[pl.ds(i * LANES, LANES)] = jnp.zeros((LANES,), jnp.float32)
        return carry

    lax.fori_loop(0, 2000 // LANES, zf, 0)

    @pl.when(s < 5)
    def _():
        pltpu.sync_copy(zbuf, acc.at[pl.ds(s * 2000, 2000)])

    plsc.subcore_barrier()

    def burst(b, carry):
        def fire(k, carry2):
            pltpu.async_copy(ew_v.at[b * 25 + k], acc.at[col_v.at[b * 25 + k]],
                             dsem, add=True)
            return carry2

        lax.fori_loop(0, 25, fire, 0)

        def drain(k, carry2):
            pltpu.make_async_copy(ew_v.at[0], acc.at[col_v.at[0]],
                                  dsem).wait()
            return carry2

        lax.fori_loop(0, 25, drain, 0)
        return carry

    lax.fori_loop(0, NCHUNK // 25, burst, 0)
    plsc.subcore_barrier()

    @pl.when(s < 5)
    def _():
        pltpu.sync_copy(acc.at[pl.ds(s * 2000, 2000)], zbuf)
        pltpu.sync_copy(zbuf, out_hbm.at[pl.ds(c * N + s * 2000, 2000)])


@functools.partial(
    pl.kernel,
    out_type=jax.ShapeDtypeStruct((NC * N, D), jnp.float32),
    mesh=_mesh,
    scratch_types=[
        pltpu.VMEM((NCHUNK, CHUNK), jnp.int32),
        pltpu.VMEM((NCHUNK, CHUNK), jnp.int32),
        pltpu.VMEM((NCHUNK, CHUNK), jnp.float32),
        pltpu.VMEM((CHUNK, D), jnp.float32),
        pltpu.VMEM((CHUNK, D), jnp.float32),
        pltpu.VMEM((NCHUNK + 3,), jnp.int32),
        pltpu.VMEM_SHARED((N, D), jnp.float32),
        pltpu.SemaphoreType.DMA,
        pltpu.SemaphoreType.DMA,
        pltpu.SemaphoreType.DMA,
        pltpu.SemaphoreType.DMA,
    ],
    compiler_params=_sc_params,
)
def _scat_kernel(hs_hbm, row_hbm, col_hbm, ew_hbm, out_hbm,
                 row_v, col_v, ew_v, gbufA, gbufB, idx_v, acc,
                 gsemA, gsemB, ssemA, ssemB):
    c = lax.axis_index("c")
    s = lax.axis_index("s")
    wid = s * NC + c

    def ifill(i, carry):
        iv = (lax.broadcasted_iota(jnp.int32, (LANES,), 0)
              + (wid * NCHUNK + i * LANES))
        idx_v[pl.ds(i * LANES, LANES)] = jnp.minimum(iv, NW * NCHUNK - 1)
        return carry

    lax.fori_loop(0, (NCHUNK + 3) // LANES, ifill, 0)
    idx125 = idx_v.at[pl.ds(0, NCHUNK)]
    pltpu.async_copy(row_hbm.at[idx125], row_v, gsemA).wait()
    pltpu.async_copy(hs_hbm.at[row_v.at[0]], gbufA, gsemA)
    pltpu.async_copy(col_hbm.at[idx125], col_v, gsemB)
    pltpu.async_copy(ew_hbm.at[idx125], ew_v, ssemA)

    def zf(r, carry):
        for cc in range(D // LANES):
            gbufB[r, pl.ds(cc * LANES, LANES)] = jnp.zeros((LANES,),
                                                           jnp.float32)
        return carry

    lax.fori_loop(0, CHUNK, zf, 0)

    @pl.when(s < WB_TILES)
    def _():
        def zc(m, carry):
            pltpu.async_copy(gbufB, acc.at[pl.ds(s * WB_ROWS + m * CHUNK,
                                                 CHUNK)], ssemB)
            return carry

        lax.fori_loop(0, WB_ROWS // CHUNK, zc, 0)
        pltpu.async_copy(gbufB.at[pl.ds(0, 40)],
                         acc.at[pl.ds(s * WB_ROWS + 960, 40)], ssemB)

        def zd(m, carry):
            pltpu.make_async_copy(
                gbufB, acc.at[pl.ds(s * WB_ROWS, CHUNK)], ssemB).wait()
            return carry

        lax.fori_loop(0, WB_ROWS // CHUNK, zd, 0)
        pltpu.make_async_copy(gbufB.at[pl.ds(0, 40)],
                              acc.at[pl.ds(s * WB_ROWS, 40)], ssemB).wait()

    pltpu.make_async_copy(col_hbm.at[idx125], col_v, gsemB).wait()
    pltpu.make_async_copy(ew_hbm.at[idx125], ew_v, ssemA).wait()
    plsc.subcore_barrier()

    dn = lax.GatherDimensionNumbers(
        offset_dims=(), collapsed_slice_dims=(0,), start_index_map=(0,))

    def scale(buf, j):
        def scale16(rb, carry):
            wch = ew_v[j, pl.ds(rb * LANES, LANES)]
            for l in range(LANES):
                wv = lax.gather(wch, jnp.full((LANES, 1), l, jnp.int32), dn,
                                slice_sizes=(1,),
                                mode=lax.GatherScatterMode.PROMISE_IN_BOUNDS)
                r = rb * LANES + l
                for cc in range(D // LANES):
                    sl = pl.ds(cc * LANES, LANES)
                    buf[r, sl] = buf[r, sl] * wv
            return carry

        lax.fori_loop(0, CHUNK // LANES, scale16, 0)

    def gather_start(j, buf, sem):
        return pltpu.async_copy(hs_hbm.at[row_v.at[j]], buf, sem)

    def gather_wait(buf, sem):
        pltpu.make_async_copy(hs_hbm.at[row_v.at[0]], buf, sem).wait()

    def scat_start(j, buf, sem):
        pltpu.async_copy(buf, acc.at[col_v.at[j]], sem, add=True)

    def scat_wait(buf, sem):
        pltpu.make_async_copy(buf, acc.at[col_v.at[0]], sem).wait()


    def body(m, carry):
        jA = 2 * m
        jB = 2 * m + 1

        @pl.when(m > 0)
        def _():
            scat_wait(gbufB, ssemB)

        gather_start(jB, gbufB, gsemB)
        gather_wait(gbufA, gsemA)
        scale(gbufA, jA)
        scat_start(jA, gbufA, ssemA)
        gather_wait(gbufB, gsemB)
        scale(gbufB, jB)
        scat_wait(gbufA, ssemA)
        gather_start(jA + 2, gbufA, gsemA)
        scat_start(jB, gbufB, ssemB)
        return carry

    lax.fori_loop(0, (NCHUNK - 1) // 2, body, 0)

    gather_wait(gbufA, gsemA)
    scale(gbufA, NCHUNK - 1)
    scat_wait(gbufB, ssemB)
    pltpu.sync_copy(gbufA, acc.at[col_v.at[NCHUNK - 1]], add=True)

    plsc.subcore_barrier()

    @pl.when(s < WB_TILES)
    def _():
        pltpu.sync_copy(acc.at[pl.ds(s * WB_ROWS, WB_ROWS)],
                        out_hbm.at[pl.ds(c * N + s * WB_ROWS, WB_ROWS)])


BN = 2000
NB = N // BN
_seq = pltpu.CompilerParams(dimension_semantics=("arbitrary",))


def _tcA(deg_a, deg_b, x, W):
    def body(da_ref, db_ref, x_ref, w_ref, dis_ref, hs_ref):
        deg = da_ref[0, 0, :] + db_ref[0, 0, :] + 1.0
        dis = jnp.where(deg > 0, lax.rsqrt(jnp.maximum(deg, 1e-12)), 0.0)
        dis_ref[0, 0, :] = dis
        h = jnp.dot(x_ref[...], w_ref[...], preferred_element_type=jnp.float32)
        hs_ref[...] = h * dis[:, None]

    return pl.pallas_call(
        body,
        grid=(NB,),
        in_specs=[
            pl.BlockSpec((1, 1, BN), lambda i: (i, 0, 0)),
            pl.BlockSpec((1, 1, BN), lambda i: (i, 0, 0)),
            pl.BlockSpec((BN, D), lambda i: (i, 0)),
            pl.BlockSpec((D, D), lambda i: (0, 0)),
        ],
        out_specs=[
            pl.BlockSpec((1, 1, BN), lambda i: (i, 0, 0)),
            pl.BlockSpec((BN, D), lambda i: (i, 0)),
        ],
        out_shape=[
            jax.ShapeDtypeStruct((NB, 1, BN), jnp.float32),
            jax.ShapeDtypeStruct((N, D), jnp.float32),
        ],
        compiler_params=_seq,
    )(deg_a, deg_b, x, W)


def _tcB(sa, sb, hs0, dis, b, gamma, beta, W):
    def body(sa_ref, sb_ref, hs_ref, dis_ref, b_ref, g_ref, be_ref, w_ref,
             out_ref):
        dis_v = dis_ref[0, 0, :]
        t = (sa_ref[...] + sb_ref[...] + hs_ref[...]) * dis_v[:, None]
        t = t + b_ref[0, :][None, :]
        t = jnp.maximum(t, 0.0)
        t = t * (g_ref[0, :] / jnp.sqrt(1.0 + EPS))[None, :] + be_ref[0, :][None, :]
        h1 = jnp.dot(t, w_ref[...], preferred_element_type=jnp.float32)
        out_ref[...] = h1 * dis_v[:, None]

    return pl.pallas_call(
        body,
        grid=(NB,),
        in_specs=[
            pl.BlockSpec((BN, D), lambda i: (i, 0)),
            pl.BlockSpec((BN, D), lambda i: (i, 0)),
            pl.BlockSpec((BN, D), lambda i: (i, 0)),
            pl.BlockSpec((1, 1, BN), lambda i: (i, 0, 0)),
            pl.BlockSpec((1, D), lambda i: (0, 0)),
            pl.BlockSpec((1, D), lambda i: (0, 0)),
            pl.BlockSpec((1, D), lambda i: (0, 0)),
            pl.BlockSpec((D, D), lambda i: (0, 0)),
        ],
        out_specs=pl.BlockSpec((BN, D), lambda i: (i, 0)),
        out_shape=jax.ShapeDtypeStruct((N, D), jnp.float32),
        compiler_params=_seq,
    )(sa, sb, hs0, dis, b, gamma, beta, W)


def _tcC(sa, sb, hs1, dis, b, gamma, beta, w1, b1, w2, b2, batch,
         cw1, cb1, cw2, cb2):
    def body(sa_ref, sb_ref, hs_ref, dis_ref, b_ref, g_ref, be_ref,
             w1_ref, b1_ref, w2_ref, b2_ref, batch_ref,
             cw1_ref, cb1_ref, cw2_ref, cb2_ref,
             h_ref, pool_ref, reps_ref, log_ref):
        i = pl.program_id(0)
        dis_v = dis_ref[0, 0, :]
        t = (sa_ref[...] + sb_ref[...] + hs_ref[...]) * dis_v[:, None]
        t = t + b_ref[0, :][None, :]
        t = jnp.maximum(t, 0.0)
        t = t * (g_ref[0, :] / jnp.sqrt(1.0 + EPS))[None, :] + be_ref[0, :][None, :]
        t1 = jnp.dot(t, w1_ref[...], preferred_element_type=jnp.float32)
        t1 = jnp.maximum(t1 + b1_ref[0, :][None, :], 0.0)
        ho = jnp.dot(t1, w2_ref[...], preferred_element_type=jnp.float32)
        ho = ho + b2_ref[0, :][None, :]
        h_ref[...] = ho
        seg_ids = lax.broadcasted_iota(jnp.int32, (G, BN), 0)
        M = (batch_ref[0, 0, :][None, :] == seg_ids).astype(jnp.float32)
        num = jnp.dot(M, ho, preferred_element_type=jnp.float32)
        cnt = jnp.sum(M, axis=1)
        blk = jnp.concatenate(
            [num, jnp.broadcast_to(cnt[:, None], (G, D))], axis=0)

        @pl.when(i == 0)
        def _():
            pool_ref[...] = jnp.zeros((2 * G, D), jnp.float32)

        pool_ref[...] += blk

        @pl.when(i == NB - 1)
        def _():
            reps = pool_ref[0:G, :] / jnp.maximum(pool_ref[G:2 * G, :], 1.0)
            reps_ref[...] = reps
            z = jnp.dot(reps, cw1_ref[...], preferred_element_type=jnp.float32)
            z = jnp.maximum(z + cb1_ref[0, :][None, :], 0.0)
            lg = jnp.dot(z, cw2_ref[...], preferred_element_type=jnp.float32)
            log_ref[...] = lg + cb2_ref[0, :][None, :]

    return pl.pallas_call(
        body,
        grid=(NB,),
        in_specs=[
            pl.BlockSpec((BN, D), lambda i: (i, 0)),
            pl.BlockSpec((BN, D), lambda i: (i, 0)),
            pl.BlockSpec((BN, D), lambda i: (i, 0)),
            pl.BlockSpec((1, 1, BN), lambda i: (i, 0, 0)),
            pl.BlockSpec((1, D), lambda i: (0, 0)),
            pl.BlockSpec((1, D), lambda i: (0, 0)),
            pl.BlockSpec((1, D), lambda i: (0, 0)),
            pl.BlockSpec((D, D), lambda i: (0, 0)),
            pl.BlockSpec((1, D), lambda i: (0, 0)),
            pl.BlockSpec((D, D), lambda i: (0, 0)),
            pl.BlockSpec((1, D), lambda i: (0, 0)),
            pl.BlockSpec((1, 1, BN), lambda i: (i, 0, 0)),
            pl.BlockSpec((D, D), lambda i: (0, 0)),
            pl.BlockSpec((1, D), lambda i: (0, 0)),
            pl.BlockSpec((D, OUT), lambda i: (0, 0)),
            pl.BlockSpec((1, OUT), lambda i: (0, 0)),
        ],
        out_specs=[
            pl.BlockSpec((BN, D), lambda i: (i, 0)),
            pl.BlockSpec((2 * G, D), lambda i: (0, 0)),
            pl.BlockSpec((G, D), lambda i: (0, 0)),
            pl.BlockSpec((G, OUT), lambda i: (0, 0)),
        ],
        out_shape=[
            jax.ShapeDtypeStruct((N, D), jnp.float32),
            jax.ShapeDtypeStruct((2 * G, D), jnp.float32),
            jax.ShapeDtypeStruct((G, D), jnp.float32),
            jax.ShapeDtypeStruct((G, OUT), jnp.float32),
        ],
        compiler_params=_seq,
    )(sa, sb, hs1, dis, b, gamma, beta, w1, b1, w2, b2, batch,
      cw1, cb1, cw2, cb2)


def kernel(x, edge_index, edge_attr, batch, W_gc0, b_gc0, gamma0, beta0,
           W_gc1, b_gc1, gamma1, beta1, lin1_W, lin1_b, lin2_W, lin2_b,
           cls1_W, cls1_b, cls2_W, cls2_b):
    ew = edge_attr[:, 0]
    row2 = edge_index[0].reshape(E // CHUNK, CHUNK)
    col2 = edge_index[1].reshape(E // CHUNK, CHUNK)
    ew2 = ew.reshape(E // CHUNK, CHUNK)

    deg2 = _deg_kernel(col2, ew2).reshape(NC, N)
    dis3, hs0 = _tcA(deg2[0].reshape(NB, 1, BN), deg2[1].reshape(NB, 1, BN),
                     x, W_gc0)
    S0 = _scat_kernel(hs0, row2, col2, ew2)
    hs1 = _tcB(S0[:N], S0[N:], hs0, dis3, b_gc0.reshape(1, D),
               gamma0.reshape(1, D), beta0.reshape(1, D), W_gc1)
    S1 = _scat_kernel(hs1, row2, col2, ew2)
    hout, _pool, reps, logits = _tcC(
        S1[:N], S1[N:], hs1, dis3, b_gc1.reshape(1, D),
        gamma1.reshape(1, D), beta1.reshape(1, D),
        lin1_W, lin1_b.reshape(1, D), lin2_W,
        lin2_b.reshape(1, D), batch.reshape(NB, 1, BN),
        cls1_W, cls1_b.reshape(1, D), cls2_W, cls2_b.reshape(1, OUT))
    return (hout, reps, logits)
